# baseline (speedup 1.0000x reference)
import functools

import jax
import jax.numpy as jnp
from jax import lax
from jax.experimental import pallas as pl
from jax.experimental.pallas import tpu as pltpu
from jax.experimental.pallas import tpu_sc as plsc

U, T, Q, K = 4096, 50, 100000, 8
TP = 64
TABW = 16
NW = 32
UPW = U // NW
PER_W = UPW * T
NP = K + 2
NCHK = 4
UH = UPW // NCHK
PH = UH * T


def _sc_gather(qid2, tab):
    mesh = plsc.VectorSubcoreMesh(core_axis_name="c", subcore_axis_name="s")

    @functools.partial(
        pl.kernel,
        mesh=mesh,
        out_type=jax.ShapeDtypeStruct((NP, U, TP), jnp.float32),
        scratch_types=[
            pltpu.VMEM((PER_W,), jnp.int32),
            pltpu.VMEM((PH, TABW), jnp.float32),
            pltpu.VMEM((PH, TABW), jnp.float32),
            pltpu.VMEM((NP, UH, TP), jnp.float32),
            pltpu.SemaphoreType.DMA,
            pltpu.SemaphoreType.DMA,
        ],
        compiler_params=pltpu.CompilerParams(use_tc_tiling_on_sc=False,
                                             needs_layout_passes=False),
    )
    def k(qid_hbm, tab_hbm, out_hbm, idx_v, rows_a, rows_b, pl_v, sem_a, sem_b):
        wid = lax.axis_index("s") * 2 + lax.axis_index("c")
        ubase = wid * UPW
        pltpu.sync_copy(qid_hbm.at[wid], idx_v)
        lanes = lax.broadcasted_iota(jnp.int32, (16,), 0)
        rowoff = [jnp.minimum(cc * 16 + lanes, T - 1) for cc in range(TP // 16)]
        bufs = [(rows_a, sem_a), (rows_b, sem_b)]

        def fire(q):
            rv, sm = bufs[q % 2]
            pltpu.async_copy(tab_hbm.at[idx_v.at[pl.ds(q * PH, PH)]], rv, sm)

        fire(0)
        fire(1)
        for q in range(NCHK):
            rows_v, sm = bufs[q % 2]
            pltpu.make_async_copy(tab_hbm.at[pl.ds(0, PH)], rows_v, sm).wait()

            def regroup(u, carry):
                rbase = u * T
                for cc in range(TP // 16):
                    row_idx = rbase + rowoff[cc]
                    for kk in range(NP):
                        v = plsc.load_gather(
                            rows_v, [row_idx, jnp.full((16,), kk, jnp.int32)])
                        pl_v[kk, u, pl.ds(cc * 16, 16)] = v
                return carry

            lax.fori_loop(0, UH, regroup, 0)
            if q + 2 < NCHK:
                fire(q + 2)
            pltpu.sync_copy(pl_v, out_hbm.at[:, pl.ds(ubase + q * UH, UH)])

    return k(qid2, tab)


def _tc_dense(planes, eps_t, Ltri):
    UB = 1024
    prec = lax.Precision.HIGHEST

    def body(pl_ref, eps_ref, l_ref, out_ref):
        Lm = l_ref[...]
        num = jnp.zeros((UB, T), jnp.float32)
        den = jnp.zeros((UB, T), jnp.float32)
        for kk in range(K):
            mk = pl_ref[kk][:, :T]
            ek = eps_ref[kk]
            yk = lax.dot(mk * ek, Lm, precision=prec,
                         preferred_element_type=jnp.float32)
            num += yk * mk
            den += mk
        dgv = pl_ref[K][:, :T]
        cgv = pl_ref[K + 1][:, :T]
        ability = num / jnp.maximum(den, 1e-8)
        out_ref[...] = cgv * (ability - dgv)

    return pl.pallas_call(
        body,
        grid=(U // UB,),
        in_specs=[
            pl.BlockSpec((NP, UB, TP), lambda i: (0, i, 0)),
            pl.BlockSpec((K, UB, T), lambda i: (0, i, 0)),
            pl.BlockSpec((T, T), lambda i: (0, 0)),
        ],
        out_specs=pl.BlockSpec((UB, T), lambda i: (i, 0)),
        out_shape=jax.ShapeDtypeStruct((U, T), jnp.float32),
        compiler_params=pltpu.CompilerParams(dimension_semantics=("arbitrary",)),
    )(planes, eps_t, Ltri)


def kernel(mask, q_id, kmap, resp, eps, diff_w, disc_w):
    tab = jnp.concatenate(
        [kmap.astype(jnp.float32), diff_w[:, None], disc_w[:, None],
         jnp.zeros((Q, TABW - K - 2), jnp.float32)], axis=1)
    qid2 = q_id.astype(jnp.int32).reshape(NW, PER_W)
    planes = _sc_gather(qid2, tab)
    eps_t = jnp.transpose(eps, (2, 0, 1))
    r = lax.broadcasted_iota(jnp.int32, (T, T), 0)
    c = lax.broadcasted_iota(jnp.int32, (T, T), 1)
    Ltri = (r <= c).astype(jnp.float32)
    return _tc_dense(planes, eps_t, Ltri)


# speedup vs baseline: 1.0050x; 1.0050x over previous
import functools

import jax
import jax.numpy as jnp
from jax import lax
from jax.experimental import pallas as pl
from jax.experimental.pallas import tpu as pltpu
from jax.experimental.pallas import tpu_sc as plsc

U, T, Q, K = 4096, 50, 100000, 8
TP = 64
TABW = 16
NW = 32
UPW = U // NW
PER_W = UPW * T
NP = K + 2
NCHK = 4
UH = UPW // NCHK
PH = UH * T


def _sc_gather(qid2, tab):
    mesh = plsc.VectorSubcoreMesh(core_axis_name="c", subcore_axis_name="s")

    @functools.partial(
        pl.kernel,
        mesh=mesh,
        out_type=jax.ShapeDtypeStruct((NP, U, TP), jnp.float32),
        scratch_types=[
            pltpu.VMEM((PER_W,), jnp.int32),
            pltpu.VMEM((PH, TABW), jnp.float32),
            pltpu.VMEM((PH, TABW), jnp.float32),
            pltpu.VMEM((NP, UH, TP), jnp.float32),
            pltpu.SemaphoreType.DMA,
            pltpu.SemaphoreType.DMA,
        ],
        compiler_params=pltpu.CompilerParams(use_tc_tiling_on_sc=False,
                                             needs_layout_passes=False),
    )
    def k(qid_hbm, tab_hbm, out_hbm, idx_v, rows_a, rows_b, pl_v, sem_a, sem_b):
        wid = lax.axis_index("s") * 2 + lax.axis_index("c")
        ubase = wid * UPW
        pltpu.sync_copy(qid_hbm.at[wid], idx_v)
        lanes = lax.broadcasted_iota(jnp.int32, (16,), 0)
        rowoff = [jnp.minimum(cc * 16 + lanes, T - 1) for cc in range(TP // 16)]
        bufs = [(rows_a, sem_a), (rows_b, sem_b)]

        def fire(q):
            rv, sm = bufs[q % 2]
            pltpu.async_copy(tab_hbm.at[idx_v.at[pl.ds(q * PH, PH)]], rv, sm)

        fire(0)
        fire(1)
        for q in range(NCHK):
            rows_v, sm = bufs[q % 2]
            pltpu.make_async_copy(tab_hbm.at[pl.ds(0, PH)], rows_v, sm).wait()

            def regroup(u, carry):
                rbase = u * T
                for cc in range(TP // 16):
                    row_idx = rbase + rowoff[cc]
                    for kk in range(NP):
                        v = plsc.load_gather(
                            rows_v, [row_idx, jnp.full((16,), kk, jnp.int32)])
                        pl_v[kk, u, pl.ds(cc * 16, 16)] = v
                return carry

            lax.fori_loop(0, UH, regroup, 0)
            if q + 2 < NCHK:
                fire(q + 2)
            pltpu.sync_copy(pl_v, out_hbm.at[:, pl.ds(ubase + q * UH, UH)])

    return k(qid2, tab)


def _tc_dense(planes, eps_t, Ltri):
    UB = 512
    prec = lax.Precision.HIGHEST

    def body(pl_ref, eps_ref, l_ref, out_ref):
        Lm = l_ref[...]
        num = jnp.zeros((UB, T), jnp.float32)
        den = jnp.zeros((UB, T), jnp.float32)
        for kk in range(K):
            mk = pl_ref[kk][:, :T]
            ek = eps_ref[kk]
            yk = lax.dot(mk * ek, Lm, precision=prec,
                         preferred_element_type=jnp.float32)
            num += yk * mk
            den += mk
        dgv = pl_ref[K][:, :T]
        cgv = pl_ref[K + 1][:, :T]
        ability = num / jnp.maximum(den, 1e-8)
        out_ref[...] = cgv * (ability - dgv)

    return pl.pallas_call(
        body,
        grid=(U // UB,),
        in_specs=[
            pl.BlockSpec((NP, UB, TP), lambda i: (0, i, 0)),
            pl.BlockSpec((K, UB, T), lambda i: (0, i, 0)),
            pl.BlockSpec((T, T), lambda i: (0, 0)),
        ],
        out_specs=pl.BlockSpec((UB, T), lambda i: (i, 0)),
        out_shape=jax.ShapeDtypeStruct((U, T), jnp.float32),
        compiler_params=pltpu.CompilerParams(dimension_semantics=("arbitrary",)),
    )(planes, eps_t, Ltri)


def kernel(mask, q_id, kmap, resp, eps, diff_w, disc_w):
    tab = jnp.concatenate(
        [kmap.astype(jnp.float32), diff_w[:, None], disc_w[:, None],
         jnp.zeros((Q, TABW - K - 2), jnp.float32)], axis=1)
    qid2 = q_id.astype(jnp.int32).reshape(NW, PER_W)
    planes = _sc_gather(qid2, tab)
    eps_t = jnp.transpose(eps, (2, 0, 1))
    r = lax.broadcasted_iota(jnp.int32, (T, T), 0)
    c = lax.broadcasted_iota(jnp.int32, (T, T), 1)
    Ltri = (r <= c).astype(jnp.float32)
    return _tc_dense(planes, eps_t, Ltri)
